# Initial kernel scaffold; baseline (speedup 1.0000x reference)
import functools

import jax
import jax.numpy as jnp
from jax import lax
from jax.experimental import pallas as pl
from jax.experimental.pallas import tpu as pltpu
from jax.experimental.pallas import tpu_sc as plsc

N_NODES = 10000
N_EDGES = 320000
NC, NS = 2, 16
NW = NC * NS
EPW = N_EDGES // NW
CHUNK = 80
NCHUNK = EPW // CHUNK
SLAB = N_NODES // NS
CNTW = 16

_MESH = plsc.VectorSubcoreMesh(
    core_axis_name="c", subcore_axis_name="s", num_cores=NC, num_subcores=NS)


def _make_seg_sum(d, with_cnt):
  out_type = [jax.ShapeDtypeStruct((NC, N_NODES, d), jnp.float32)]
  scratch = [
      pltpu.VMEM((NCHUNK, CHUNK), jnp.int32),
      pltpu.VMEM((NCHUNK, CHUNK), jnp.int32),
      pltpu.VMEM((CHUNK, d), jnp.float32),
      pltpu.VMEM_SHARED((N_NODES, d), jnp.float32),
      pltpu.SemaphoreType.DMA,
  ]
  if with_cnt:
    out_type.append(jax.ShapeDtypeStruct((NC, N_NODES, CNTW), jnp.float32))
    scratch += [
        pltpu.VMEM((CHUNK, CNTW), jnp.float32),
        pltpu.VMEM_SHARED((N_NODES, CNTW), jnp.float32),
    ]

  def body(*refs):
    if with_cnt:
      (table, srcw, dstw, zrows, zcnt, ones_hbm,
       out, cnt_out, idx_s, idx_d, rows, acc_sh, sem, ones_v, cnt_sh) = refs
    else:
      (table, srcw, dstw, zrows,
       out, idx_s, idx_d, rows, acc_sh, sem) = refs
    c = lax.axis_index("c")
    s = lax.axis_index("s")
    wid = c * NS + s
    slab = pl.ds(s * SLAB, SLAB)

    pltpu.sync_copy(zrows, acc_sh.at[slab])
    pltpu.sync_copy(srcw.at[wid], idx_s)
    pltpu.sync_copy(dstw.at[wid], idx_d)
    if with_cnt:
      pltpu.sync_copy(zcnt, cnt_sh.at[slab])
      pltpu.sync_copy(ones_hbm, ones_v)
    plsc.subcore_barrier()

    def step(g, carry):
      pltpu.async_copy(table.at[idx_s.at[g]], rows, sem).wait()
      pltpu.sync_copy(rows, acc_sh.at[idx_d.at[g]], add=True)
      if with_cnt:
        pltpu.sync_copy(ones_v, cnt_sh.at[idx_d.at[g]], add=True)
      return carry

    lax.fori_loop(0, NCHUNK, step, 0)

    plsc.subcore_barrier()
    pltpu.sync_copy(acc_sh.at[slab], out.at[c, slab])
    if with_cnt:
      pltpu.sync_copy(cnt_sh.at[slab], cnt_out.at[c, slab])

  return pl.kernel(body, out_type=out_type, mesh=_MESH, scratch_types=scratch,
                   name=f"seg_sum_d{d}" + ("_cnt" if with_cnt else ""))


_seg128_cnt = _make_seg_sum(128, True)
_seg128 = _make_seg_sum(128, False)
_seg64 = _make_seg_sum(64, False)


def _inv_deg(cntp_ref):
  cnt = cntp_ref[0, :, 0:1] + cntp_ref[1, :, 0:1]
  return 1.0 / jnp.maximum(cnt, 1.0)


def _dot_t(a, w):
  return lax.dot_general(a, w, (((1,), (1,)), ((), ())),
                         preferred_element_type=jnp.float32)


_NB = 1000


def _tc1_body(aggp, cntp, x, w1l, b1l, w1r, h1):
  agg = (aggp[0] + aggp[1]) * _inv_deg(cntp)
  h = _dot_t(agg, w1l[...]) + b1l[...] + _dot_t(x[...], w1r[...])
  h1[...] = jnp.maximum(h, 0.0)


def _tc2_body(aggp, cntp, h1, w2l, b2l, w2r, w3l, h2, z):
  agg = (aggp[0] + aggp[1]) * _inv_deg(cntp)
  h = _dot_t(agg, w2l[...]) + b2l[...] + _dot_t(h1[...], w2r[...])
  h = jnp.maximum(h, 0.0)
  h2[...] = h
  z[...] = _dot_t(h, w3l[...])


def _tc3_body(aggp, cntp, h2, w3r, b3l, out):
  agg = (aggp[0] + aggp[1]) * _inv_deg(cntp)
  out[...] = agg + b3l[...] + _dot_t(h2[...], w3r[...])


def _row_spec(d):
  return pl.BlockSpec((_NB, d), lambda i: (i, 0))


def _part_spec(d):
  return pl.BlockSpec((NC, _NB, d), lambda i: (0, i, 0))


def _full_spec(shape):
  return pl.BlockSpec(shape, lambda i: tuple(0 for _ in shape))


_GRID = N_NODES // _NB

_tc1 = pl.pallas_call(
    _tc1_body,
    grid=(_GRID,),
    in_specs=[_part_spec(128), _part_spec(CNTW), _row_spec(128),
              _full_spec((128, 128)), _full_spec((1, 128)),
              _full_spec((128, 128))],
    out_specs=_row_spec(128),
    out_shape=jax.ShapeDtypeStruct((N_NODES, 128), jnp.float32),
)

_tc2 = pl.pallas_call(
    _tc2_body,
    grid=(_GRID,),
    in_specs=[_part_spec(128), _part_spec(CNTW), _row_spec(128),
              _full_spec((256, 128)), _full_spec((1, 256)),
              _full_spec((256, 128)), _full_spec((64, 256))],
    out_specs=[_row_spec(256), _row_spec(64)],
    out_shape=[jax.ShapeDtypeStruct((N_NODES, 256), jnp.float32),
               jax.ShapeDtypeStruct((N_NODES, 64), jnp.float32)],
)

_tc3 = pl.pallas_call(
    _tc3_body,
    grid=(_GRID,),
    in_specs=[_part_spec(64), _part_spec(CNTW), _row_spec(256),
              _full_spec((64, 256)), _full_spec((1, 64))],
    out_specs=_row_spec(64),
    out_shape=jax.ShapeDtypeStruct((N_NODES, 64), jnp.float32),
)


@jax.jit
def kernel(x, edge_index, W1l, b1l, W1r, W2l, b2l, W2r, W3l, b3l, W3r):
  src = edge_index[0].astype(jnp.int32).reshape(NW, NCHUNK, CHUNK)
  dst = edge_index[1].astype(jnp.int32).reshape(NW, NCHUNK, CHUNK)
  z128 = jnp.zeros((SLAB, 128), jnp.float32)
  z64 = jnp.zeros((SLAB, 64), jnp.float32)
  zcnt = jnp.zeros((SLAB, CNTW), jnp.float32)
  ones = jnp.ones((CHUNK, CNTW), jnp.float32)

  aggp1, cntp = _seg128_cnt(x, src, dst, z128, zcnt, ones)
  h1 = _tc1(aggp1, cntp, x, W1l, b1l.reshape(1, -1), W1r)
  aggp2 = _seg128(h1, src, dst, z128)
  h2, z = _tc2(aggp2, cntp, h1, W2l, b2l.reshape(1, -1), W2r, W3l)
  aggp3 = _seg64(z, src, dst, z64)
  return _tc3(aggp3, cntp, h2, W3r, b3l.reshape(1, -1))


# speedup vs baseline: 8.7621x; 8.7621x over previous
import functools

import jax
import jax.numpy as jnp
from jax import lax
from jax.experimental import pallas as pl
from jax.experimental.pallas import tpu as pltpu
from jax.experimental.pallas import tpu_sc as plsc

N_NODES = 10000
N_EDGES = 320000
NC, NS = 2, 16
NW = NC * NS
EPW = N_EDGES // NW
CHUNK = 80
NCHUNK = EPW // CHUNK
N_PAD = 10240
SLAB = N_PAD // NS
CNTW = 16

_MESH = plsc.VectorSubcoreMesh(
    core_axis_name="c", subcore_axis_name="s", num_cores=NC, num_subcores=NS)


def _make_seg_sum(d):
  out_type = [jax.ShapeDtypeStruct((NC, N_PAD, d), jnp.float32)]
  scratch = [
      pltpu.VMEM((NCHUNK, CHUNK), jnp.int32),
      pltpu.VMEM((NCHUNK, CHUNK), jnp.int32),
      pltpu.VMEM((CHUNK, d), jnp.float32),
      pltpu.VMEM_SHARED((N_PAD, d), jnp.float32),
      pltpu.SemaphoreType.DMA,
  ]

  def body(table, srcw, dstw, zrows, out, idx_s, idx_d, rows, acc_sh, sem):
    c = lax.axis_index("c")
    s = lax.axis_index("s")
    wid = c * NS + s
    slab = pl.ds(s * SLAB, SLAB)

    pltpu.sync_copy(zrows, acc_sh.at[slab])
    pltpu.sync_copy(srcw.at[wid], idx_s)
    pltpu.sync_copy(dstw.at[wid], idx_d)
    plsc.subcore_barrier()

    def step(g, carry):
      pltpu.async_copy(table.at[idx_s.at[g]], rows, sem).wait()
      pltpu.sync_copy(rows, acc_sh.at[idx_d.at[g]], add=True)
      return carry

    lax.fori_loop(0, NCHUNK, step, 0)

    plsc.subcore_barrier()
    pltpu.sync_copy(acc_sh.at[slab], out.at[c, slab])

  params = None
  if d % 128 != 0:
    params = pltpu.CompilerParams(use_tc_tiling_on_sc=False)
  return pl.kernel(body, out_type=out_type, mesh=_MESH, scratch_types=scratch,
                   compiler_params=params, name=f"seg_sum_d{d}")


def _make_cnt():
  out_type = [jax.ShapeDtypeStruct((NC, N_PAD, CNTW), jnp.float32)]
  scratch = [
      pltpu.VMEM((NCHUNK, CHUNK), jnp.int32),
      pltpu.VMEM((CHUNK, CNTW), jnp.float32),
      pltpu.VMEM_SHARED((N_PAD, CNTW), jnp.float32),
  ]

  def body(dstw, zcnt, ones_hbm, cnt_out, idx_d, ones_v, cnt_sh):
    c = lax.axis_index("c")
    s = lax.axis_index("s")
    wid = c * NS + s
    slab = pl.ds(s * SLAB, SLAB)

    pltpu.sync_copy(zcnt, cnt_sh.at[slab])
    pltpu.sync_copy(dstw.at[wid], idx_d)
    pltpu.sync_copy(ones_hbm, ones_v)
    plsc.subcore_barrier()

    def step(g, carry):
      pltpu.sync_copy(ones_v, cnt_sh.at[idx_d.at[g]], add=True)
      return carry

    lax.fori_loop(0, NCHUNK, step, 0)

    plsc.subcore_barrier()
    pltpu.sync_copy(cnt_sh.at[slab], cnt_out.at[c, slab])

  return pl.kernel(body, out_type=out_type, mesh=_MESH, scratch_types=scratch,
                   compiler_params=pltpu.CompilerParams(use_tc_tiling_on_sc=False),
                   name="deg_cnt")


_seg128 = _make_seg_sum(128)
_seg64 = _make_seg_sum(64)
_cnt = _make_cnt()


def _inv_deg(cntp_ref):
  cnt = cntp_ref[0, :, 0:1] + cntp_ref[1, :, 0:1]
  return 1.0 / jnp.maximum(cnt, 1.0)


def _dot_t(a, w):
  return lax.dot_general(a, w, (((1,), (1,)), ((), ())),
                         preferred_element_type=jnp.float32)


_NB = 1000


def _tc1_body(aggp, cntp, x, w1l, b1l, w1r, h1):
  agg = (aggp[0] + aggp[1]) * _inv_deg(cntp)
  h = _dot_t(agg, w1l[...]) + b1l[...] + _dot_t(x[...], w1r[...])
  h1[...] = jnp.maximum(h, 0.0)


def _tc2_body(aggp, cntp, h1, w2l, b2l, w2r, w3l, h2, z):
  agg = (aggp[0] + aggp[1]) * _inv_deg(cntp)
  h = _dot_t(agg, w2l[...]) + b2l[...] + _dot_t(h1[...], w2r[...])
  h = jnp.maximum(h, 0.0)
  h2[...] = h
  z[...] = _dot_t(h, w3l[...])


def _tc3_body(aggp, cntp, h2, w3r, b3l, out):
  agg = (aggp[0] + aggp[1]) * _inv_deg(cntp)
  out[...] = agg + b3l[...] + _dot_t(h2[...], w3r[...])


def _row_spec(d):
  return pl.BlockSpec((_NB, d), lambda i: (i, 0))


def _part_spec(d):
  return pl.BlockSpec((NC, _NB, d), lambda i: (0, i, 0))


def _full_spec(shape):
  return pl.BlockSpec(shape, lambda i: tuple(0 for _ in shape))


_GRID = N_NODES // _NB

_tc1 = pl.pallas_call(
    _tc1_body,
    grid=(_GRID,),
    in_specs=[_part_spec(128), _part_spec(CNTW), _row_spec(128),
              _full_spec((128, 128)), _full_spec((1, 128)),
              _full_spec((128, 128))],
    out_specs=_row_spec(128),
    out_shape=jax.ShapeDtypeStruct((N_NODES, 128), jnp.float32),
)

_tc2 = pl.pallas_call(
    _tc2_body,
    grid=(_GRID,),
    in_specs=[_part_spec(128), _part_spec(CNTW), _row_spec(128),
              _full_spec((256, 128)), _full_spec((1, 256)),
              _full_spec((256, 128)), _full_spec((64, 256))],
    out_specs=[_row_spec(256), _row_spec(64)],
    out_shape=[jax.ShapeDtypeStruct((N_NODES, 256), jnp.float32),
               jax.ShapeDtypeStruct((N_NODES, 64), jnp.float32)],
)

_tc3 = pl.pallas_call(
    _tc3_body,
    grid=(_GRID,),
    in_specs=[_part_spec(64), _part_spec(CNTW), _row_spec(256),
              _full_spec((64, 256)), _full_spec((1, 64))],
    out_specs=_row_spec(64),
    out_shape=jax.ShapeDtypeStruct((N_NODES, 64), jnp.float32),
)


@jax.jit
def kernel(x, edge_index, W1l, b1l, W1r, W2l, b2l, W2r, W3l, b3l, W3r):
  src = edge_index[0].astype(jnp.int32).reshape(NW, NCHUNK, CHUNK)
  dst = edge_index[1].astype(jnp.int32).reshape(NW, NCHUNK, CHUNK)
  z128 = jnp.zeros((SLAB, 128), jnp.float32)
  z64 = jnp.zeros((SLAB, 64), jnp.float32)
  zcnt = jnp.zeros((SLAB, CNTW), jnp.float32)
  ones = jnp.ones((CHUNK, CNTW), jnp.float32)

  cntp, = _cnt(dst, zcnt, ones)
  aggp1, = _seg128(x, src, dst, z128)
  h1 = _tc1(aggp1, cntp, x, W1l, b1l.reshape(1, -1), W1r)
  aggp2, = _seg128(h1, src, dst, z128)
  h2, z = _tc2(aggp2, cntp, h1, W2l, b2l.reshape(1, -1), W2r, W3l)
  aggp3, = _seg64(z, src, dst, z64)
  return _tc3(aggp3, cntp, h2, W3r, b3l.reshape(1, -1))


# speedup vs baseline: 11.2268x; 1.2813x over previous
import jax
import jax.numpy as jnp
from jax import lax
from jax.experimental import pallas as pl
from jax.experimental.pallas import tpu as pltpu
from jax.experimental.pallas import tpu_sc as plsc

N_NODES = 10000
N_EDGES = 320000
NC, NS = 2, 16
NW = NC * NS
CHUNK = 80
NCH_A = N_EDGES // NS // CHUNK
NCH_B = N_EDGES // NW // CHUNK
N_PAD = 10240
SLAB = N_PAD // NS
CNTW = 16
DC = 64

_MESH = plsc.VectorSubcoreMesh(
    core_axis_name="c", subcore_axis_name="s", num_cores=NC, num_subcores=NS)


def _pipeline(table_of, idx_s, idx_d, rows0, rows1, acc_sh, sem0, sem1,
              nchunk):
  def gstart(g, buf, sem):
    pltpu.async_copy(table_of(idx_s.at[g]), buf, sem)

  def gwait(buf, sem):
    pltpu.make_async_copy(table_of(idx_s.at[0]), buf, sem).wait()

  def scat(g, buf):
    pltpu.sync_copy(buf, acc_sh.at[idx_d.at[g]], add=True)

  gstart(0, rows0, sem0)

  def step(g, carry):
    gstart(2 * g + 1, rows1, sem1)
    gwait(rows0, sem0)
    scat(2 * g, rows0)
    gstart(2 * g + 2, rows0, sem0)
    gwait(rows1, sem1)
    scat(2 * g + 1, rows1)
    return carry

  if nchunk % 2:
    lax.fori_loop(0, (nchunk - 1) // 2, step, 0)
    gwait(rows0, sem0)
    scat(nchunk - 1, rows0)
  else:
    lax.fori_loop(0, nchunk // 2 - 1, step, 0)
    gstart(nchunk - 1, rows1, sem1)
    gwait(rows0, sem0)
    scat(nchunk - 2, rows0)
    gwait(rows1, sem1)
    scat(nchunk - 1, rows1)


def _zero_acc(rows0, acc_sh, s, d):
  zv = jnp.zeros((16,), jnp.float32)
  vpr = d // 16
  shift = vpr.bit_length() - 1

  def zstore(i, carry):
    rows0[i >> shift, pl.ds((i & (vpr - 1)) * 16, 16)] = zv
    return carry

  lax.fori_loop(0, CHUNK * vpr, zstore, 0)
  for t in range(SLAB // CHUNK):
    pltpu.sync_copy(rows0, acc_sh.at[pl.ds(s * SLAB + t * CHUNK, CHUNK)])


def _make_seg_colsplit():
  out_type = [jax.ShapeDtypeStruct((NC, N_PAD, DC), jnp.float32)]
  scratch = [
      pltpu.VMEM((NCH_A, CHUNK), jnp.int32),
      pltpu.VMEM((NCH_A, CHUNK), jnp.int32),
      pltpu.VMEM((CHUNK, DC), jnp.float32),
      pltpu.VMEM((CHUNK, DC), jnp.float32),
      pltpu.VMEM_SHARED((N_PAD, DC), jnp.float32),
      pltpu.SemaphoreType.DMA,
      pltpu.SemaphoreType.DMA,
  ]

  def body(table, srcw, dstw, out, idx_s, idx_d, rows0, rows1,
           acc_sh, sem0, sem1):
    c = lax.axis_index("c")
    s = lax.axis_index("s")
    slab = pl.ds(s * SLAB, SLAB)

    pltpu.sync_copy(srcw.at[s], idx_s)
    pltpu.sync_copy(dstw.at[s], idx_d)
    _zero_acc(rows0, acc_sh, s, DC)
    plsc.subcore_barrier()

    _pipeline(lambda i: table.at[c].at[i], idx_s, idx_d, rows0, rows1,
              acc_sh, sem0, sem1, NCH_A)

    plsc.subcore_barrier()
    pltpu.sync_copy(acc_sh.at[slab], out.at[c, slab])

  params = pltpu.CompilerParams(use_tc_tiling_on_sc=False)
  return pl.kernel(body, out_type=out_type, mesh=_MESH, scratch_types=scratch,
                   compiler_params=params, name="seg_colsplit")


def _make_seg_rowsplit():
  out_type = [jax.ShapeDtypeStruct((NC, N_PAD, DC), jnp.float32)]
  scratch = [
      pltpu.VMEM((NCH_B, CHUNK), jnp.int32),
      pltpu.VMEM((NCH_B, CHUNK), jnp.int32),
      pltpu.VMEM((CHUNK, DC), jnp.float32),
      pltpu.VMEM((CHUNK, DC), jnp.float32),
      pltpu.VMEM_SHARED((N_PAD, DC), jnp.float32),
      pltpu.SemaphoreType.DMA,
      pltpu.SemaphoreType.DMA,
  ]

  def body(table, srcw, dstw, out, idx_s, idx_d, rows0, rows1,
           acc_sh, sem0, sem1):
    c = lax.axis_index("c")
    s = lax.axis_index("s")
    wid = c * NS + s
    slab = pl.ds(s * SLAB, SLAB)

    pltpu.sync_copy(srcw.at[wid], idx_s)
    pltpu.sync_copy(dstw.at[wid], idx_d)
    _zero_acc(rows0, acc_sh, s, DC)
    plsc.subcore_barrier()

    _pipeline(lambda i: table.at[i], idx_s, idx_d, rows0, rows1,
              acc_sh, sem0, sem1, NCH_B)

    plsc.subcore_barrier()
    pltpu.sync_copy(acc_sh.at[slab], out.at[c, slab])

  params = pltpu.CompilerParams(use_tc_tiling_on_sc=False)
  return pl.kernel(body, out_type=out_type, mesh=_MESH, scratch_types=scratch,
                   compiler_params=params, name="seg_rowsplit")


def _make_cnt():
  out_type = [jax.ShapeDtypeStruct((NC, N_PAD, CNTW), jnp.float32)]
  scratch = [
      pltpu.VMEM((NCH_B, CHUNK), jnp.int32),
      pltpu.VMEM((CHUNK, CNTW), jnp.float32),
      pltpu.VMEM_SHARED((N_PAD, CNTW), jnp.float32),
  ]

  def body(dstw, zcnt, ones_hbm, cnt_out, idx_d, ones_v, cnt_sh):
    c = lax.axis_index("c")
    s = lax.axis_index("s")
    wid = c * NS + s
    slab = pl.ds(s * SLAB, SLAB)

    pltpu.sync_copy(zcnt, cnt_sh.at[slab])
    pltpu.sync_copy(dstw.at[wid], idx_d)
    pltpu.sync_copy(ones_hbm, ones_v)
    plsc.subcore_barrier()

    def step(g, carry):
      pltpu.sync_copy(ones_v, cnt_sh.at[idx_d.at[g]], add=True)
      return carry

    lax.fori_loop(0, NCH_B, step, 0)

    plsc.subcore_barrier()
    pltpu.sync_copy(cnt_sh.at[slab], cnt_out.at[c, slab])

  return pl.kernel(body, out_type=out_type, mesh=_MESH, scratch_types=scratch,
                   compiler_params=pltpu.CompilerParams(use_tc_tiling_on_sc=False),
                   name="deg_cnt")


_seg_col = _make_seg_colsplit()
_seg_row = _make_seg_rowsplit()
_cnt = _make_cnt()


def _inv_deg(cntp_ref):
  cnt = cntp_ref[0, :, 0:1] + cntp_ref[1, :, 0:1]
  return 1.0 / jnp.maximum(cnt, 1.0)


def _dot_t(a, w):
  return lax.dot_general(a, w, (((1,), (1,)), ((), ())),
                         preferred_element_type=jnp.float32)


_NB = 1000


def _tc1_body(aggp, cntp, x, w1l, b1l, w1r, h1s):
  agg = jnp.concatenate([aggp[0], aggp[1]], axis=1) * _inv_deg(cntp)
  h = _dot_t(agg, w1l[...]) + b1l[...] + _dot_t(x[...], w1r[...])
  h = jnp.maximum(h, 0.0)
  h1s[0] = h[:, :DC]
  h1s[1] = h[:, DC:]


def _tc2_body(aggp, cntp, h1s, w2l, b2l, w2r, w3l, h2, z):
  agg = jnp.concatenate([aggp[0], aggp[1]], axis=1) * _inv_deg(cntp)
  h1 = jnp.concatenate([h1s[0], h1s[1]], axis=1)
  h = _dot_t(agg, w2l[...]) + b2l[...] + _dot_t(h1, w2r[...])
  h = jnp.maximum(h, 0.0)
  h2[...] = h
  z[...] = _dot_t(h, w3l[...])


def _tc3_body(aggp, cntp, h2, w3r, b3l, out):
  agg = (aggp[0] + aggp[1]) * _inv_deg(cntp)
  out[...] = agg + b3l[...] + _dot_t(h2[...], w3r[...])


def _row_spec(d):
  return pl.BlockSpec((_NB, d), lambda i: (i, 0))


def _part_spec(d):
  return pl.BlockSpec((NC, _NB, d), lambda i: (0, i, 0))


def _full_spec(shape):
  return pl.BlockSpec(shape, lambda i: tuple(0 for _ in shape))


_GRID = N_NODES // _NB

_tc1 = pl.pallas_call(
    _tc1_body,
    grid=(_GRID,),
    in_specs=[_part_spec(DC), _part_spec(CNTW), _row_spec(128),
              _full_spec((128, 128)), _full_spec((1, 128)),
              _full_spec((128, 128))],
    out_specs=_part_spec(DC),
    out_shape=jax.ShapeDtypeStruct((NC, N_NODES, DC), jnp.float32),
)

_tc2 = pl.pallas_call(
    _tc2_body,
    grid=(_GRID,),
    in_specs=[_part_spec(DC), _part_spec(CNTW), _part_spec(DC),
              _full_spec((256, 128)), _full_spec((1, 256)),
              _full_spec((256, 128)), _full_spec((64, 256))],
    out_specs=[_row_spec(256), _row_spec(64)],
    out_shape=[jax.ShapeDtypeStruct((N_NODES, 256), jnp.float32),
               jax.ShapeDtypeStruct((N_NODES, 64), jnp.float32)],
)

_tc3 = pl.pallas_call(
    _tc3_body,
    grid=(_GRID,),
    in_specs=[_part_spec(64), _part_spec(CNTW), _row_spec(256),
              _full_spec((64, 256)), _full_spec((1, 64))],
    out_specs=_row_spec(64),
    out_shape=jax.ShapeDtypeStruct((N_NODES, 64), jnp.float32),
)


@jax.jit
def kernel(x, edge_index, W1l, b1l, W1r, W2l, b2l, W2r, W3l, b3l, W3r):
  src = edge_index[0].astype(jnp.int32)
  dst = edge_index[1].astype(jnp.int32)
  src16 = src.reshape(NS, NCH_A, CHUNK)
  dst16 = dst.reshape(NS, NCH_A, CHUNK)
  src32 = src.reshape(NW, NCH_B, CHUNK)
  dst32 = dst.reshape(NW, NCH_B, CHUNK)
  zcnt = jnp.zeros((SLAB, CNTW), jnp.float32)
  ones = jnp.ones((CHUNK, CNTW), jnp.float32)

  xs = jnp.stack([x[:, :DC], x[:, DC:]])
  cntp, = _cnt(dst32, zcnt, ones)
  aggp1, = _seg_col(xs, src16, dst16)
  h1s = _tc1(aggp1, cntp, x, W1l, b1l.reshape(1, -1), W1r)
  aggp2, = _seg_col(h1s, src16, dst16)
  h2, z = _tc2(aggp2, cntp, h1s, W2l, b2l.reshape(1, -1), W2r, W3l)
  aggp3, = _seg_row(z, src32, dst32)
  return _tc3(aggp3, cntp, h2, W3r, b3l.reshape(1, -1))
